# baseline (device time: 162337 ns/iter reference)
import jax
import jax.numpy as jnp
from jax import lax
from jax.experimental import pallas as pl
from jax.experimental.pallas import tpu as pltpu

C = 32


def kernel(x):
    m, n = x.shape
    M = 2 * m
    half = m // 2
    ch = half // C

    def body(x_hbm, out_ref, xtmp, ltmp_sem, y_send, y_recv, x_send, x_recv):
        my_x = lax.axis_index("x")
        my_y = lax.axis_index("y")

        barrier = pltpu.get_barrier_semaphore()
        pl.semaphore_signal(barrier, inc=1, device_id=(my_x, 1 - my_y),
                            device_id_type=pl.DeviceIdType.MESH)
        pl.semaphore_signal(barrier, inc=1, device_id=(1 - my_x, my_y),
                            device_id_type=pl.DeviceIdType.MESH)
        pl.semaphore_wait(barrier, 2)

        def load(src_row, slot):
            return pltpu.make_async_copy(
                x_hbm.at[pl.ds(src_row, ch), :], xtmp.at[slot],
                ltmp_sem.at[slot])

        own0 = my_y * m + my_x * half
        fwd0 = (1 - my_y) * m + my_x * half

        def rdma_y(c):
            sl = pl.ds(own0 + c * ch, ch)
            return pltpu.make_async_remote_copy(
                src_ref=out_ref.at[sl, :], dst_ref=out_ref.at[sl, :],
                send_sem=y_send.at[c], recv_sem=y_recv.at[c],
                device_id=(my_x, 1 - my_y),
                device_id_type=pl.DeviceIdType.MESH)

        def rdma_x(c):
            sl = pl.ds(fwd0 + c * ch, ch)
            return pltpu.make_async_remote_copy(
                src_ref=out_ref.at[sl, :], dst_ref=out_ref.at[sl, :],
                send_sem=x_send.at[c], recv_sem=x_recv.at[c],
                device_id=(1 - my_x, my_y),
                device_id_type=pl.DeviceIdType.MESH)

        def stage_chunk(src_half_row, dst_global_row, c):
            load(src_half_row + c * ch, c % 2).wait()
            if c + 1 < C:
                load(src_half_row + (c + 1) * ch, (c + 1) % 2).start()
            out_ref[pl.ds(dst_global_row + c * ch, ch), :] = (
                xtmp[c % 2].astype(out_ref.dtype))

        my_src = my_x * half
        load(my_src, 0).start()
        for c in range(C):
            stage_chunk(my_src, own0, c)
            rdma_y(c).start()

        oth_src = (1 - my_x) * half
        oth0 = my_y * m + (1 - my_x) * half
        load(oth_src, 0).start()
        for c in range(C):
            rdma_y(c).wait_recv()
            rdma_x(c).start()
            stage_chunk(oth_src, oth0, c)

        for c in range(C):
            rdma_x(c).wait_recv()
        for c in range(C):
            rdma_y(c).wait_send()
            rdma_x(c).wait_send()

    return pl.pallas_call(
        body,
        out_shape=jax.ShapeDtypeStruct((M, n), jnp.bfloat16),
        in_specs=[pl.BlockSpec(memory_space=pl.ANY)],
        out_specs=pl.BlockSpec(memory_space=pltpu.VMEM),
        scratch_shapes=[
            pltpu.VMEM((2, ch, n), x.dtype),
            pltpu.SemaphoreType.DMA((2,)),
            pltpu.SemaphoreType.DMA((C,)),
            pltpu.SemaphoreType.DMA((C,)),
            pltpu.SemaphoreType.DMA((C,)),
            pltpu.SemaphoreType.DMA((C,)),
        ],
        compiler_params=pltpu.CompilerParams(
            collective_id=0, vmem_limit_bytes=60 * 1024 * 1024),
    )(x)


# device time: 141648 ns/iter; 1.1461x vs baseline; 1.1461x over previous
import jax
import jax.numpy as jnp
from jax import lax
from jax.experimental import pallas as pl
from jax.experimental.pallas import tpu as pltpu

CC = 32
CS = 8
R = CC // CS
_P1_BLOCKS = [(0, 128)] + [(128 + 512 * i, 512) for i in range(7)] + [(3712, 384)]


def kernel(x):
    m, n = x.shape
    M = 2 * m
    half = m // 2
    ch = half // CC
    chs = half // CS
    assert [sum(c for _, c in _P1_BLOCKS), _P1_BLOCKS[-1][0] + _P1_BLOCKS[-1][1]] == [half, half]

    def body(x_hbm, out_ref, xtmp, ltmp_sem, y_send, y_recv, x_send, x_recv):
        my_x = lax.axis_index("x")
        my_y = lax.axis_index("y")

        barrier = pltpu.get_barrier_semaphore()
        pl.semaphore_signal(barrier, inc=1, device_id=(my_x, 1 - my_y),
                            device_id_type=pl.DeviceIdType.MESH)
        pl.semaphore_signal(barrier, inc=1, device_id=(1 - my_x, my_y),
                            device_id_type=pl.DeviceIdType.MESH)
        pl.semaphore_wait(barrier, 2)

        own0 = my_y * m + my_x * half
        fwd0 = (1 - my_y) * m + my_x * half

        def rdma_y(c):
            sl = pl.ds(own0 + c * ch, ch)
            return pltpu.make_async_remote_copy(
                src_ref=out_ref.at[sl, :], dst_ref=out_ref.at[sl, :],
                send_sem=y_send.at[c], recv_sem=y_recv.at[c],
                device_id=(my_x, 1 - my_y),
                device_id_type=pl.DeviceIdType.MESH)

        def rdma_x(c):
            sl = pl.ds(fwd0 + c * ch, ch)
            return pltpu.make_async_remote_copy(
                src_ref=out_ref.at[sl, :], dst_ref=out_ref.at[sl, :],
                send_sem=x_send.at[c], recv_sem=x_recv.at[c],
                device_id=(1 - my_x, my_y),
                device_id_type=pl.DeviceIdType.MESH)

        my_src = my_x * half

        def p1_load(b, slot):
            r0, cnt = _P1_BLOCKS[b]
            return pltpu.make_async_copy(
                x_hbm.at[pl.ds(my_src + r0, cnt), :],
                xtmp.at[slot, pl.ds(0, cnt), :], ltmp_sem.at[slot])

        p1_load(0, 0).start()
        sent = 0
        for b, (r0, cnt) in enumerate(_P1_BLOCKS):
            p1_load(b, b % 2).wait()
            if b + 1 < len(_P1_BLOCKS):
                p1_load(b + 1, (b + 1) % 2).start()
            out_ref[pl.ds(own0 + r0, cnt), :] = (
                xtmp[b % 2, pl.ds(0, cnt), :].astype(out_ref.dtype))
            while (sent + 1) * ch <= r0 + cnt:
                rdma_y(sent).start()
                sent += 1

        oth_src = (1 - my_x) * half
        oth0 = my_y * m + (1 - my_x) * half

        def p2_load(b, slot):
            return pltpu.make_async_copy(
                x_hbm.at[pl.ds(oth_src + b * chs, chs), :],
                xtmp.at[slot], ltmp_sem.at[slot])

        p2_load(0, 0).start()
        for c in range(CC):
            rdma_y(c).wait_recv()
            rdma_x(c).start()
            b = c // R
            if c % R == 0:
                p2_load(b, b % 2).wait()
                if b + 1 < CS:
                    p2_load(b + 1, (b + 1) % 2).start()
            out_ref[pl.ds(oth0 + c * ch, ch), :] = (
                xtmp[b % 2, pl.ds((c % R) * ch, ch), :].astype(out_ref.dtype))

        for c in range(CC):
            rdma_x(c).wait_recv()
        for c in range(CC):
            rdma_y(c).wait_send()
            rdma_x(c).wait_send()

    return pl.pallas_call(
        body,
        out_shape=jax.ShapeDtypeStruct((M, n), jnp.bfloat16),
        in_specs=[pl.BlockSpec(memory_space=pl.ANY)],
        out_specs=pl.BlockSpec(memory_space=pltpu.VMEM),
        scratch_shapes=[
            pltpu.VMEM((2, half // CS, n), x.dtype),
            pltpu.SemaphoreType.DMA((2,)),
            pltpu.SemaphoreType.DMA((CC,)),
            pltpu.SemaphoreType.DMA((CC,)),
            pltpu.SemaphoreType.DMA((CC,)),
            pltpu.SemaphoreType.DMA((CC,)),
        ],
        compiler_params=pltpu.CompilerParams(
            collective_id=0, vmem_limit_bytes=60 * 1024 * 1024),
    )(x)


# device time: 138000 ns/iter; 1.1764x vs baseline; 1.0264x over previous
import jax
import jax.numpy as jnp
from jax import lax
from jax.experimental import pallas as pl
from jax.experimental.pallas import tpu as pltpu

CC = 32
CS = 4
R = CC // CS
_P1_BLOCKS = [(0, 128)] + [(128 + 1024 * i, 1024) for i in range(3)] + [(3200, 896)]


def kernel(x):
    m, n = x.shape
    M = 2 * m
    half = m // 2
    ch = half // CC
    chs = half // CS
    assert [sum(c for _, c in _P1_BLOCKS), _P1_BLOCKS[-1][0] + _P1_BLOCKS[-1][1]] == [half, half]

    def body(x_hbm, out_ref, xtmp, ltmp_sem, y_send, y_recv, x_send, x_recv):
        my_x = lax.axis_index("x")
        my_y = lax.axis_index("y")

        barrier = pltpu.get_barrier_semaphore()
        pl.semaphore_signal(barrier, inc=1, device_id=(my_x, 1 - my_y),
                            device_id_type=pl.DeviceIdType.MESH)
        pl.semaphore_signal(barrier, inc=1, device_id=(1 - my_x, my_y),
                            device_id_type=pl.DeviceIdType.MESH)
        pl.semaphore_wait(barrier, 2)

        own0 = my_y * m + my_x * half
        fwd0 = (1 - my_y) * m + my_x * half

        def rdma_y(c):
            sl = pl.ds(own0 + c * ch, ch)
            return pltpu.make_async_remote_copy(
                src_ref=out_ref.at[sl, :], dst_ref=out_ref.at[sl, :],
                send_sem=y_send.at[c], recv_sem=y_recv.at[c],
                device_id=(my_x, 1 - my_y),
                device_id_type=pl.DeviceIdType.MESH)

        def rdma_x(c):
            sl = pl.ds(fwd0 + c * ch, ch)
            return pltpu.make_async_remote_copy(
                src_ref=out_ref.at[sl, :], dst_ref=out_ref.at[sl, :],
                send_sem=x_send.at[c], recv_sem=x_recv.at[c],
                device_id=(1 - my_x, my_y),
                device_id_type=pl.DeviceIdType.MESH)

        my_src = my_x * half

        def p1_load(b, slot):
            r0, cnt = _P1_BLOCKS[b]
            return pltpu.make_async_copy(
                x_hbm.at[pl.ds(my_src + r0, cnt), :],
                xtmp.at[slot, pl.ds(0, cnt), :], ltmp_sem.at[slot])

        p1_load(0, 0).start()
        sent = 0
        for b, (r0, cnt) in enumerate(_P1_BLOCKS):
            p1_load(b, b % 2).wait()
            if b + 1 < len(_P1_BLOCKS):
                p1_load(b + 1, (b + 1) % 2).start()
            out_ref[pl.ds(own0 + r0, cnt), :] = (
                xtmp[b % 2, pl.ds(0, cnt), :].astype(out_ref.dtype))
            while (sent + 1) * ch <= r0 + cnt:
                rdma_y(sent).start()
                sent += 1

        oth_src = (1 - my_x) * half
        oth0 = my_y * m + (1 - my_x) * half

        def p2_load(b, slot):
            return pltpu.make_async_copy(
                x_hbm.at[pl.ds(oth_src + b * chs, chs), :],
                xtmp.at[slot], ltmp_sem.at[slot])

        p2_load(0, 0).start()
        for c in range(CC):
            rdma_y(c).wait_recv()
            rdma_x(c).start()
            b = c // R
            if c % R == 0:
                p2_load(b, b % 2).wait()
                if b + 1 < CS:
                    p2_load(b + 1, (b + 1) % 2).start()
            out_ref[pl.ds(oth0 + c * ch, ch), :] = (
                xtmp[b % 2, pl.ds((c % R) * ch, ch), :].astype(out_ref.dtype))

        for c in range(CC):
            rdma_x(c).wait_recv()
        for c in range(CC):
            rdma_y(c).wait_send()
            rdma_x(c).wait_send()

    return pl.pallas_call(
        body,
        out_shape=jax.ShapeDtypeStruct((M, n), jnp.bfloat16),
        in_specs=[pl.BlockSpec(memory_space=pl.ANY)],
        out_specs=pl.BlockSpec(memory_space=pltpu.VMEM),
        scratch_shapes=[
            pltpu.VMEM((2, half // CS, n), x.dtype),
            pltpu.SemaphoreType.DMA((2,)),
            pltpu.SemaphoreType.DMA((CC,)),
            pltpu.SemaphoreType.DMA((CC,)),
            pltpu.SemaphoreType.DMA((CC,)),
            pltpu.SemaphoreType.DMA((CC,)),
        ],
        compiler_params=pltpu.CompilerParams(
            collective_id=0, vmem_limit_bytes=60 * 1024 * 1024),
    )(x)


# device time: 137186 ns/iter; 1.1833x vs baseline; 1.0059x over previous
import jax
import jax.numpy as jnp
from jax import lax
from jax.experimental import pallas as pl
from jax.experimental.pallas import tpu as pltpu

CC = 32
CS = 2
R = CC // CS
_P1_BLOCKS = [(0, 128), (128, 2048), (2176, 1920)]


def kernel(x):
    m, n = x.shape
    M = 2 * m
    half = m // 2
    ch = half // CC
    chs = half // CS
    assert [sum(c for _, c in _P1_BLOCKS), _P1_BLOCKS[-1][0] + _P1_BLOCKS[-1][1]] == [half, half]

    def body(x_hbm, out_ref, xtmp, ltmp_sem, y_send, y_recv, x_send, x_recv):
        my_x = lax.axis_index("x")
        my_y = lax.axis_index("y")

        barrier = pltpu.get_barrier_semaphore()
        pl.semaphore_signal(barrier, inc=1, device_id=(my_x, 1 - my_y),
                            device_id_type=pl.DeviceIdType.MESH)
        pl.semaphore_signal(barrier, inc=1, device_id=(1 - my_x, my_y),
                            device_id_type=pl.DeviceIdType.MESH)
        pl.semaphore_wait(barrier, 2)

        own0 = my_y * m + my_x * half
        fwd0 = (1 - my_y) * m + my_x * half

        def rdma_y(c):
            sl = pl.ds(own0 + c * ch, ch)
            return pltpu.make_async_remote_copy(
                src_ref=out_ref.at[sl, :], dst_ref=out_ref.at[sl, :],
                send_sem=y_send.at[c], recv_sem=y_recv.at[c],
                device_id=(my_x, 1 - my_y),
                device_id_type=pl.DeviceIdType.MESH)

        def rdma_x(c):
            sl = pl.ds(fwd0 + c * ch, ch)
            return pltpu.make_async_remote_copy(
                src_ref=out_ref.at[sl, :], dst_ref=out_ref.at[sl, :],
                send_sem=x_send.at[c], recv_sem=x_recv.at[c],
                device_id=(1 - my_x, my_y),
                device_id_type=pl.DeviceIdType.MESH)

        my_src = my_x * half

        def p1_load(b, slot):
            r0, cnt = _P1_BLOCKS[b]
            return pltpu.make_async_copy(
                x_hbm.at[pl.ds(my_src + r0, cnt), :],
                xtmp.at[slot, pl.ds(0, cnt), :], ltmp_sem.at[slot])

        p1_load(0, 0).start()
        sent = 0
        for b, (r0, cnt) in enumerate(_P1_BLOCKS):
            p1_load(b, b % 2).wait()
            if b + 1 < len(_P1_BLOCKS):
                p1_load(b + 1, (b + 1) % 2).start()
            out_ref[pl.ds(own0 + r0, cnt), :] = (
                xtmp[b % 2, pl.ds(0, cnt), :].astype(out_ref.dtype))
            while (sent + 1) * ch <= r0 + cnt:
                rdma_y(sent).start()
                sent += 1

        oth_src = (1 - my_x) * half
        oth0 = my_y * m + (1 - my_x) * half

        def p2_load(b, slot):
            return pltpu.make_async_copy(
                x_hbm.at[pl.ds(oth_src + b * chs, chs), :],
                xtmp.at[slot], ltmp_sem.at[slot])

        p2_load(0, 0).start()
        for c in range(CC):
            rdma_y(c).wait_recv()
            rdma_x(c).start()
            b = c // R
            if c % R == 0:
                p2_load(b, b % 2).wait()
                if b + 1 < CS:
                    p2_load(b + 1, (b + 1) % 2).start()
            out_ref[pl.ds(oth0 + c * ch, ch), :] = (
                xtmp[b % 2, pl.ds((c % R) * ch, ch), :].astype(out_ref.dtype))

        for c in range(CC):
            rdma_x(c).wait_recv()
        for c in range(CC):
            rdma_y(c).wait_send()
            rdma_x(c).wait_send()

    return pl.pallas_call(
        body,
        out_shape=jax.ShapeDtypeStruct((M, n), jnp.bfloat16),
        in_specs=[pl.BlockSpec(memory_space=pl.ANY)],
        out_specs=pl.BlockSpec(memory_space=pltpu.VMEM),
        scratch_shapes=[
            pltpu.VMEM((2, half // CS, n), x.dtype),
            pltpu.SemaphoreType.DMA((2,)),
            pltpu.SemaphoreType.DMA((CC,)),
            pltpu.SemaphoreType.DMA((CC,)),
            pltpu.SemaphoreType.DMA((CC,)),
            pltpu.SemaphoreType.DMA((CC,)),
        ],
        compiler_params=pltpu.CompilerParams(
            collective_id=0, vmem_limit_bytes=60 * 1024 * 1024),
    )(x)
